# baseline (speedup 1.0000x reference)
import functools

import jax
import jax.numpy as jnp
from jax import lax
from jax.experimental import pallas as pl
from jax.experimental.pallas import tpu as pltpu
from jax.experimental.pallas import tpu_sc as plsc

_E = 64
_CAPF = 1.25
_TPS = 2048
_LO_BITS = 0x3C000000
_HI_BITS = 0x3F800000


def _phase1_body(hs_ref, wt_ref, w_ref, u_ref, e_ref, psum_ref, cnt_ref, aux_ref,
                 *, grid):
    x = hs_ref[...]
    wt = wt_ref[...]
    logits = jnp.dot(x, wt, preferred_element_type=jnp.float32)
    m = jnp.max(logits, axis=1, keepdims=True)
    ex = jnp.exp(logits - m)
    s = jnp.sum(ex, axis=1, keepdims=True)
    wmax = 1.0 / s
    c, e = logits.shape
    iota_e = lax.broadcasted_iota(jnp.int32, (c, e), 1)
    eidx = jnp.min(jnp.where(logits == m, iota_e, e), axis=1, keepdims=True)
    w_ref[...] = wmax
    u_ref[...] = lax.bitcast_convert_type(wmax, jnp.int32)
    e_ref[...] = eidx
    probs = ex * wmax
    psum_part = jnp.sum(probs, axis=0, keepdims=True)
    onehot = (iota_e == eidx).astype(jnp.float32)
    cnt_part = jnp.sum(onehot, axis=0, keepdims=True)

    @pl.when(pl.program_id(0) == 0)
    def _init():
        psum_ref[...] = jnp.zeros_like(psum_ref)
        cnt_ref[...] = jnp.zeros_like(cnt_ref)

    psum_ref[...] += jnp.broadcast_to(psum_part, psum_ref.shape)
    cnt_ref[...] += jnp.broadcast_to(cnt_part, cnt_ref.shape)

    n = c * grid

    @pl.when(pl.program_id(0) == grid - 1)
    def _aux():
        aux = jnp.sum(cnt_ref[0:1, :] * psum_ref[0:1, :], axis=1,
                      keepdims=True)
        aux_ref[...] = aux * (_E / (n * float(n)))


def _sc_body(w_hbm, u_hbm, e_hbm, out_hbm,
             wv, ev, uv, sidx, cnt_tab, cnt_loc, slab_loc,
             lo, hi, mid, tt, ii, slots, stage, slab, *, cap, n):
    cid = lax.axis_index("c")
    sid = lax.axis_index("s")
    base = sid * _TPS
    nv = _TPS // 16
    pltpu.sync_copy(w_hbm.at[pl.ds(base, _TPS)], wv)
    pltpu.sync_copy(u_hbm.at[pl.ds(base, _TPS)], uv)
    pltpu.sync_copy(e_hbm.at[pl.ds(base, _TPS)], ev)
    lane = lax.iota(jnp.int32, 16)
    ones = jnp.ones((16,), jnp.int32)

    def prep(j, _):
        sl = pl.ds(j * 16, 16)
        sidx[sl] = lane * _E + ev[sl]
        return 0

    lax.fori_loop(0, nv, prep, 0)

    def init1(j, _):
        sl = pl.ds(j * 16, 16)
        lo[sl] = jnp.full((16,), _LO_BITS, jnp.int32)
        hi[sl] = jnp.full((16,), _HI_BITS, jnp.int32)
        mid[sl] = jnp.full((16,), _LO_BITS + ((_HI_BITS - _LO_BITS) >> 1),
                           jnp.int32)
        return 0

    lax.fori_loop(0, 4, init1, 0)

    def scan_count(pred):
        def z(j, _):
            cnt_tab[pl.ds(j * 16, 16)] = jnp.zeros((16,), jnp.int32)
            return 0

        lax.fori_loop(0, _E, z, 0)

        def sc(j, _):
            sl = pl.ds(j * 16, 16)
            bit = pred(uv[sl], ev[sl], base + j * 16 + lane)
            plsc.addupdate_scatter(cnt_tab, [sidx[sl]], ones, mask=bit)
            return 0

        lax.fori_loop(0, nv, sc, 0)

        def red(j, _):
            acc = jnp.zeros((16,), jnp.int32)
            for l in range(16):
                acc = acc + cnt_tab[pl.ds(l * _E + j * 16, 16)]
            cnt_loc[pl.ds(j * 16, 16)] = acc
            return 0

        lax.fori_loop(0, 4, red, 0)

    def publish_reduce(par):
        pltpu.sync_copy(cnt_loc, slab.at[pl.ds((par * 16 + sid) * _E, _E)])
        plsc.subcore_barrier()
        pltpu.sync_copy(slab.at[pl.ds(par * 16 * _E, 16 * _E)], slab_loc)

        def red2(j, _):
            acc = jnp.zeros((16,), jnp.int32)
            for r in range(16):
                acc = acc + slab_loc[pl.ds(r * _E + j * 16, 16)]
            cnt_loc[pl.ds(j * 16, 16)] = acc
            return 0

        lax.fori_loop(0, 4, red2, 0)

    def pass1(p, _):
        scan_count(lambda u, e, gi: u >= plsc.load_gather(mid, [e]))
        publish_reduce(p & 1)

        def upd(j, _):
            sl = pl.ds(j * 16, 16)
            ok = cnt_loc[sl] >= cap
            l2 = jnp.where(ok, mid[sl], lo[sl])
            h2 = jnp.where(ok, hi[sl], mid[sl])
            lo[sl] = l2
            hi[sl] = h2
            mid[sl] = l2 + ((h2 - l2) >> 1)
            return 0

        lax.fori_loop(0, 4, upd, 0)
        return 0

    lax.fori_loop(0, 25, pass1, 0)

    def sett(j, _):
        sl = pl.ds(j * 16, 16)
        tv = lo[sl]
        tt[sl] = jnp.where(tv == _LO_BITS, 0, tv)
        return 0

    lax.fori_loop(0, 4, sett, 0)

    scan_count(lambda u, e, gi: u > plsc.load_gather(tt, [e]))
    publish_reduce(1)

    def init2(j, _):
        sl = pl.ds(j * 16, 16)
        slots[sl] = cap - cnt_loc[sl]
        lo[sl] = jnp.zeros((16,), jnp.int32)
        hi[sl] = jnp.full((16,), 65536, jnp.int32)
        mid[sl] = jnp.full((16,), 32768, jnp.int32)
        return 0

    lax.fori_loop(0, 4, init2, 0)

    def pass2(p, _):
        scan_count(lambda u, e, gi:
                   (u == plsc.load_gather(tt, [e]))
                   & (gi < plsc.load_gather(mid, [e])))
        publish_reduce(p & 1)

        def upd(j, _):
            sl = pl.ds(j * 16, 16)
            ok = cnt_loc[sl] <= slots[sl]
            l2 = jnp.where(ok, mid[sl], lo[sl])
            h2 = jnp.where(ok, hi[sl], mid[sl])
            lo[sl] = l2
            hi[sl] = h2
            mid[sl] = l2 + ((h2 - l2) >> 1)
            return 0

        lax.fori_loop(0, 4, upd, 0)
        return 0

    lax.fori_loop(0, 17, pass2, 0)

    def seti(j, _):
        sl = pl.ds(j * 16, 16)
        ii[sl] = lo[sl]
        return 0

    lax.fori_loop(0, 4, seti, 0)

    lbase = cid * (_TPS // 2)

    def app(j, _):
        sl = pl.ds(lbase + j * 16, 16)
        u = uv[sl]
        e = ev[sl]
        tg = plsc.load_gather(tt, [e])
        ig = plsc.load_gather(ii, [e])
        gi = base + lbase + j * 16 + lane
        keep = (u > tg) | ((u == tg) & (gi < ig))
        stage[pl.ds(j * 16, 16)] = jnp.where(keep, wv[sl], 0.0)
        return 0

    lax.fori_loop(0, _TPS // 32, app, 0)
    pltpu.sync_copy(stage, out_hbm.at[pl.ds(base + lbase, _TPS // 2)])


def _sc_select(w1, u1, e1, n, cap):
    mesh = plsc.VectorSubcoreMesh(core_axis_name="c", subcore_axis_name="s")
    k = functools.partial(
        pl.kernel,
        mesh=mesh,
        compiler_params=pltpu.CompilerParams(needs_layout_passes=False),
        out_type=jax.ShapeDtypeStruct((n,), jnp.float32),
        scratch_types=[
            pltpu.VMEM((_TPS,), jnp.float32),
            pltpu.VMEM((_TPS,), jnp.int32),
            pltpu.VMEM((_TPS,), jnp.int32),
            pltpu.VMEM((_TPS,), jnp.int32),
            pltpu.VMEM((16 * _E,), jnp.int32),
            pltpu.VMEM((_E,), jnp.int32),
            pltpu.VMEM((16 * _E,), jnp.int32),
            pltpu.VMEM((_E,), jnp.int32),
            pltpu.VMEM((_E,), jnp.int32),
            pltpu.VMEM((_E,), jnp.int32),
            pltpu.VMEM((_E,), jnp.int32),
            pltpu.VMEM((_E,), jnp.int32),
            pltpu.VMEM((_E,), jnp.int32),
            pltpu.VMEM((_TPS // 2,), jnp.float32),
            pltpu.VMEM_SHARED((2 * 16 * _E,), jnp.int32),
        ],
    )(functools.partial(_sc_body, cap=cap, n=n))
    return k(w1, u1, e1)


def kernel(hidden_states, W_gate):
    b, s, d = hidden_states.shape
    n = b * s
    e = W_gate.shape[0]
    cap = int(n * _CAPF / e)
    chunk = 2048
    grid = n // chunk
    hs2 = hidden_states.reshape(n, d)
    wt = W_gate.T

    w1, u1, e1, psum, cnt, aux = pl.pallas_call(
        functools.partial(_phase1_body, grid=grid),
        grid=(grid,),
        in_specs=[
            pl.BlockSpec((chunk, d), lambda i: (i, 0)),
            pl.BlockSpec((d, e), lambda i: (0, 0)),
        ],
        out_specs=[
            pl.BlockSpec((chunk, 1), lambda i: (i, 0)),
            pl.BlockSpec((chunk, 1), lambda i: (i, 0)),
            pl.BlockSpec((chunk, 1), lambda i: (i, 0)),
            pl.BlockSpec((8, e), lambda i: (0, 0)),
            pl.BlockSpec((8, e), lambda i: (0, 0)),
            pl.BlockSpec((1, 1), lambda i: (0, 0)),
        ],
        out_shape=[
            jax.ShapeDtypeStruct((n, 1), jnp.float32),
            jax.ShapeDtypeStruct((n, 1), jnp.int32),
            jax.ShapeDtypeStruct((n, 1), jnp.int32),
            jax.ShapeDtypeStruct((8, e), jnp.float32),
            jax.ShapeDtypeStruct((8, e), jnp.float32),
            jax.ShapeDtypeStruct((1, 1), jnp.float32),
        ],
    )(hs2, wt)

    wk = _sc_select(w1.reshape(n), u1.reshape(n), e1.reshape(n), n, cap)

    return (wk.reshape(n, 1), e1, cnt[0], aux[0, 0])


# speedup vs baseline: 1.3155x; 1.3155x over previous
import functools

import jax
import jax.numpy as jnp
from jax import lax
from jax.experimental import pallas as pl
from jax.experimental.pallas import tpu as pltpu
from jax.experimental.pallas import tpu_sc as plsc

_E = 64
_CAPF = 1.25
_TPS = 2048
_LO_BITS = 0x3C000000
_HI_BITS = 0x3F800000


def _phase1_body(hs_ref, wt_ref, w_ref, u_ref, e_ref, psum_ref, cnt_ref, aux_ref,
                 *, grid):
    x = hs_ref[...]
    wt = wt_ref[...]
    logits = jnp.dot(x, wt, preferred_element_type=jnp.float32)
    m = jnp.max(logits, axis=1, keepdims=True)
    ex = jnp.exp(logits - m)
    s = jnp.sum(ex, axis=1, keepdims=True)
    wmax = 1.0 / s
    c, e = logits.shape
    iota_e = lax.broadcasted_iota(jnp.int32, (c, e), 1)
    eidx = jnp.min(jnp.where(logits == m, iota_e, e), axis=1, keepdims=True)
    w_ref[...] = wmax
    u_ref[...] = lax.bitcast_convert_type(wmax, jnp.int32)
    e_ref[...] = eidx
    probs = ex * wmax
    psum_part = jnp.sum(probs, axis=0, keepdims=True)
    onehot = (iota_e == eidx).astype(jnp.float32)
    cnt_part = jnp.sum(onehot, axis=0, keepdims=True)

    @pl.when(pl.program_id(0) == 0)
    def _init():
        psum_ref[...] = jnp.zeros_like(psum_ref)
        cnt_ref[...] = jnp.zeros_like(cnt_ref)

    psum_ref[...] += jnp.broadcast_to(psum_part, psum_ref.shape)
    cnt_ref[...] += jnp.broadcast_to(cnt_part, cnt_ref.shape)

    n = c * grid

    @pl.when(pl.program_id(0) == grid - 1)
    def _aux():
        aux = jnp.sum(cnt_ref[0:1, :] * psum_ref[0:1, :], axis=1,
                      keepdims=True)
        aux_ref[...] = aux * (_E / (n * float(n)))


def _sc_body(w_hbm, u_hbm, e_hbm, out_hbm,
             wv, ev, uv, sidx, cnt_tab, cnt_loc, slab_loc,
             lo, hi, mid, tt, ii, slots, stage, slab, *, cap, n):
    cid = lax.axis_index("c")
    sid = lax.axis_index("s")
    base = sid * _TPS
    nv = _TPS // 16
    pltpu.sync_copy(w_hbm.at[pl.ds(base, _TPS)], wv)
    pltpu.sync_copy(u_hbm.at[pl.ds(base, _TPS)], uv)
    pltpu.sync_copy(e_hbm.at[pl.ds(base, _TPS)], ev)
    lane = lax.iota(jnp.int32, 16)
    ones = jnp.ones((16,), jnp.int32)

    def prep(j, _):
        sl = pl.ds(j * 16, 16)
        sidx[sl] = lane * _E + ev[sl]
        return 0

    lax.fori_loop(0, nv, prep, 0)

    def init1(j, _):
        sl = pl.ds(j * 16, 16)
        lo[sl] = jnp.full((16,), _LO_BITS, jnp.int32)
        hi[sl] = jnp.full((16,), _HI_BITS, jnp.int32)
        mid[sl] = jnp.full((16,), _LO_BITS + ((_HI_BITS - _LO_BITS) >> 1),
                           jnp.int32)
        return 0

    lax.fori_loop(0, 4, init1, 0)

    def scan_count(pred):
        zero16 = jnp.zeros((16,), jnp.int32)

        def z(j, _):
            for k in range(4):
                cnt_tab[pl.ds(j * 64 + k * 16, 16)] = zero16
            return 0

        lax.fori_loop(0, _E // 4, z, 0)

        def sc(j, _):
            for k in range(4):
                sl = pl.ds(j * 64 + k * 16, 16)
                bit = pred(uv[sl], ev[sl], base + j * 64 + k * 16 + lane)
                plsc.addupdate_scatter(cnt_tab, [sidx[sl]], ones, mask=bit)
            return 0

        lax.fori_loop(0, nv // 4, sc, 0)

        def red(j, _):
            acc = jnp.zeros((16,), jnp.int32)
            for l in range(16):
                acc = acc + cnt_tab[pl.ds(l * _E + j * 16, 16)]
            cnt_loc[pl.ds(j * 16, 16)] = acc
            return 0

        lax.fori_loop(0, 4, red, 0)

    def publish_reduce(par):
        pltpu.sync_copy(cnt_loc, slab.at[pl.ds((par * 16 + sid) * _E, _E)])
        plsc.subcore_barrier()
        pltpu.sync_copy(slab.at[pl.ds(par * 16 * _E, 16 * _E)], slab_loc)

        def red2(j, _):
            acc = jnp.zeros((16,), jnp.int32)
            for r in range(16):
                acc = acc + slab_loc[pl.ds(r * _E + j * 16, 16)]
            cnt_loc[pl.ds(j * 16, 16)] = acc
            return 0

        lax.fori_loop(0, 4, red2, 0)

    def pass1(p, _):
        scan_count(lambda u, e, gi: u >= plsc.load_gather(mid, [e]))
        publish_reduce(p & 1)

        def upd(j, _):
            sl = pl.ds(j * 16, 16)
            ok = cnt_loc[sl] >= cap
            l2 = jnp.where(ok, mid[sl], lo[sl])
            h2 = jnp.where(ok, hi[sl], mid[sl])
            lo[sl] = l2
            hi[sl] = h2
            mid[sl] = l2 + ((h2 - l2) >> 1)
            return 0

        lax.fori_loop(0, 4, upd, 0)
        return 0

    lax.fori_loop(0, 25, pass1, 0)

    def sett(j, _):
        sl = pl.ds(j * 16, 16)
        tv = lo[sl]
        tt[sl] = jnp.where(tv == _LO_BITS, 0, tv)
        return 0

    lax.fori_loop(0, 4, sett, 0)

    scan_count(lambda u, e, gi: u > plsc.load_gather(tt, [e]))
    publish_reduce(1)

    def init2(j, _):
        sl = pl.ds(j * 16, 16)
        slots[sl] = cap - cnt_loc[sl]
        lo[sl] = jnp.zeros((16,), jnp.int32)
        hi[sl] = jnp.full((16,), 65536, jnp.int32)
        mid[sl] = jnp.full((16,), 32768, jnp.int32)
        return 0

    lax.fori_loop(0, 4, init2, 0)

    scan_count(lambda u, e, gi: u == plsc.load_gather(tt, [e]))
    publish_reduce(0)

    def chk(j, acc):
        sl = pl.ds(j * 16, 16)
        return jnp.maximum(acc, jnp.max(cnt_loc[sl] - slots[sl]))

    need = lax.fori_loop(0, 4, chk, jnp.int32(-(1 << 30)))

    def do_search2():
        def pass2(p, _):
            scan_count(lambda u, e, gi:
                       (u == plsc.load_gather(tt, [e]))
                       & (gi < plsc.load_gather(mid, [e])))
            publish_reduce((p + 1) & 1)

            def upd(j, _):
                sl = pl.ds(j * 16, 16)
                ok = cnt_loc[sl] <= slots[sl]
                l2 = jnp.where(ok, mid[sl], lo[sl])
                h2 = jnp.where(ok, hi[sl], mid[sl])
                lo[sl] = l2
                hi[sl] = h2
                mid[sl] = l2 + ((h2 - l2) >> 1)
                return 0

            lax.fori_loop(0, 4, upd, 0)
            return 0

        lax.fori_loop(0, 17, pass2, 0)

        def seti(j, _):
            sl = pl.ds(j * 16, 16)
            ii[sl] = lo[sl]
            return 0

        lax.fori_loop(0, 4, seti, 0)

    def skip_search2():
        full = jnp.full((16,), 65536, jnp.int32)

        def seti(j, _):
            ii[pl.ds(j * 16, 16)] = full
            return 0

        lax.fori_loop(0, 4, seti, 0)

    lax.cond(need > 0, do_search2, skip_search2)

    lbase = cid * (_TPS // 2)

    def app(j, _):
        sl = pl.ds(lbase + j * 16, 16)
        u = uv[sl]
        e = ev[sl]
        tg = plsc.load_gather(tt, [e])
        ig = plsc.load_gather(ii, [e])
        gi = base + lbase + j * 16 + lane
        keep = (u > tg) | ((u == tg) & (gi < ig))
        stage[pl.ds(j * 16, 16)] = jnp.where(keep, wv[sl], 0.0)
        return 0

    lax.fori_loop(0, _TPS // 32, app, 0)
    pltpu.sync_copy(stage, out_hbm.at[pl.ds(base + lbase, _TPS // 2)])


def _sc_select(w1, u1, e1, n, cap):
    mesh = plsc.VectorSubcoreMesh(core_axis_name="c", subcore_axis_name="s")
    k = functools.partial(
        pl.kernel,
        mesh=mesh,
        compiler_params=pltpu.CompilerParams(needs_layout_passes=False),
        out_type=jax.ShapeDtypeStruct((n,), jnp.float32),
        scratch_types=[
            pltpu.VMEM((_TPS,), jnp.float32),
            pltpu.VMEM((_TPS,), jnp.int32),
            pltpu.VMEM((_TPS,), jnp.int32),
            pltpu.VMEM((_TPS,), jnp.int32),
            pltpu.VMEM((16 * _E,), jnp.int32),
            pltpu.VMEM((_E,), jnp.int32),
            pltpu.VMEM((16 * _E,), jnp.int32),
            pltpu.VMEM((_E,), jnp.int32),
            pltpu.VMEM((_E,), jnp.int32),
            pltpu.VMEM((_E,), jnp.int32),
            pltpu.VMEM((_E,), jnp.int32),
            pltpu.VMEM((_E,), jnp.int32),
            pltpu.VMEM((_E,), jnp.int32),
            pltpu.VMEM((_TPS // 2,), jnp.float32),
            pltpu.VMEM_SHARED((2 * 16 * _E,), jnp.int32),
        ],
    )(functools.partial(_sc_body, cap=cap, n=n))
    return k(w1, u1, e1)


def kernel(hidden_states, W_gate):
    b, s, d = hidden_states.shape
    n = b * s
    e = W_gate.shape[0]
    cap = int(n * _CAPF / e)
    chunk = 2048
    grid = n // chunk
    hs2 = hidden_states.reshape(n, d)
    wt = W_gate.T

    w1, u1, e1, psum, cnt, aux = pl.pallas_call(
        functools.partial(_phase1_body, grid=grid),
        grid=(grid,),
        in_specs=[
            pl.BlockSpec((chunk, d), lambda i: (i, 0)),
            pl.BlockSpec((d, e), lambda i: (0, 0)),
        ],
        out_specs=[
            pl.BlockSpec((chunk, 1), lambda i: (i, 0)),
            pl.BlockSpec((chunk, 1), lambda i: (i, 0)),
            pl.BlockSpec((chunk, 1), lambda i: (i, 0)),
            pl.BlockSpec((8, e), lambda i: (0, 0)),
            pl.BlockSpec((8, e), lambda i: (0, 0)),
            pl.BlockSpec((1, 1), lambda i: (0, 0)),
        ],
        out_shape=[
            jax.ShapeDtypeStruct((n, 1), jnp.float32),
            jax.ShapeDtypeStruct((n, 1), jnp.int32),
            jax.ShapeDtypeStruct((n, 1), jnp.int32),
            jax.ShapeDtypeStruct((8, e), jnp.float32),
            jax.ShapeDtypeStruct((8, e), jnp.float32),
            jax.ShapeDtypeStruct((1, 1), jnp.float32),
        ],
    )(hs2, wt)

    wk = _sc_select(w1.reshape(n), u1.reshape(n), e1.reshape(n), n, cap)

    return (wk.reshape(n, 1), e1, cnt[0], aux[0, 0])


# speedup vs baseline: 1.8575x; 1.4120x over previous
import functools

import jax
import jax.numpy as jnp
from jax import lax
from jax.experimental import pallas as pl
from jax.experimental.pallas import tpu as pltpu
from jax.experimental.pallas import tpu_sc as plsc

_E = 64
_CAPF = 1.25
_TPS = 2048
_LO_BITS = 0x3C000000
_HI_BITS = 0x3F800000


def _phase1_body(hs_ref, wt_ref, w_ref, u_ref, e_ref, psum_ref, cnt_ref, aux_ref,
                 *, grid):
    x = hs_ref[...]
    wg = wt_ref[...]
    logits = lax.dot_general(wg, x, (((1,), (1,)), ((), ())),
                             preferred_element_type=jnp.float32)
    m = jnp.max(logits, axis=0, keepdims=True)
    ex = jnp.exp(logits - m)
    s = jnp.sum(ex, axis=0, keepdims=True)
    wmax = 1.0 / s
    e, c = logits.shape
    iota_e = lax.broadcasted_iota(jnp.int32, (e, c), 0)
    eidx = jnp.min(jnp.where(logits == m, iota_e, e), axis=0, keepdims=True)
    w_ref[...] = wmax.reshape(1, 1, c)
    u_ref[...] = lax.bitcast_convert_type(wmax, jnp.int32).reshape(1, 1, c)
    e_ref[...] = eidx.reshape(1, 1, c)
    probs = ex * wmax
    psum_part = jnp.sum(probs, axis=1, keepdims=True)
    onehot = (iota_e == eidx).astype(jnp.float32)
    cnt_part = jnp.sum(onehot, axis=1, keepdims=True)

    @pl.when(pl.program_id(0) == 0)
    def _init():
        psum_ref[...] = jnp.zeros_like(psum_ref)
        cnt_ref[...] = jnp.zeros_like(cnt_ref)

    psum_ref[...] += jnp.broadcast_to(psum_part, psum_ref.shape)
    cnt_ref[...] += jnp.broadcast_to(cnt_part, cnt_ref.shape)

    n = c * grid

    @pl.when(pl.program_id(0) == grid - 1)
    def _aux():
        prod = jnp.sum(cnt_ref[:, 0:1] * psum_ref[:, 0:1], axis=0,
                       keepdims=True)
        aux_ref[...] = prod * (_E / (n * float(n)))


def _sc_body(w_hbm, u_hbm, e_hbm, out_hbm,
             wv, ev, uv, sidx, cnt_tab, cnt_loc, slab_loc,
             lo, hi, mid, tt, ii, slots, stage, slab, *, cap, n):
    cid = lax.axis_index("c")
    sid = lax.axis_index("s")
    base = sid * _TPS
    nv = _TPS // 16
    pltpu.sync_copy(w_hbm.at[pl.ds(base, _TPS)], wv)
    pltpu.sync_copy(u_hbm.at[pl.ds(base, _TPS)], uv)
    pltpu.sync_copy(e_hbm.at[pl.ds(base, _TPS)], ev)
    lane = lax.iota(jnp.int32, 16)
    ones = jnp.ones((16,), jnp.int32)

    def prep(j, _):
        sl = pl.ds(j * 16, 16)
        sidx[sl] = lane * _E + ev[sl]
        return 0

    lax.fori_loop(0, nv, prep, 0)

    def init1(j, _):
        sl = pl.ds(j * 16, 16)
        lo[sl] = jnp.full((16,), _LO_BITS, jnp.int32)
        hi[sl] = jnp.full((16,), _HI_BITS, jnp.int32)
        mid[sl] = jnp.full((16,), _LO_BITS + ((_HI_BITS - _LO_BITS) >> 1),
                           jnp.int32)
        return 0

    lax.fori_loop(0, 4, init1, 0)

    def scan_count(pred):
        zero16 = jnp.zeros((16,), jnp.int32)

        def z(j, _):
            for k in range(4):
                cnt_tab[pl.ds(j * 64 + k * 16, 16)] = zero16
            return 0

        lax.fori_loop(0, _E // 4, z, 0)

        def sc(j, _):
            for k in range(4):
                sl = pl.ds(j * 64 + k * 16, 16)
                bit = pred(uv[sl], ev[sl], base + j * 64 + k * 16 + lane)
                plsc.addupdate_scatter(cnt_tab, [sidx[sl]], ones, mask=bit)
            return 0

        lax.fori_loop(0, nv // 4, sc, 0)

        def red(j, _):
            acc = jnp.zeros((16,), jnp.int32)
            for l in range(16):
                acc = acc + cnt_tab[pl.ds(l * _E + j * 16, 16)]
            cnt_loc[pl.ds(j * 16, 16)] = acc
            return 0

        lax.fori_loop(0, 4, red, 0)

    def publish_reduce(par):
        pltpu.sync_copy(cnt_loc, slab.at[pl.ds((par * 16 + sid) * _E, _E)])
        plsc.subcore_barrier()
        pltpu.sync_copy(slab.at[pl.ds(par * 16 * _E, 16 * _E)], slab_loc)

        def red2(j, _):
            acc = jnp.zeros((16,), jnp.int32)
            for r in range(16):
                acc = acc + slab_loc[pl.ds(r * _E + j * 16, 16)]
            cnt_loc[pl.ds(j * 16, 16)] = acc
            return 0

        lax.fori_loop(0, 4, red2, 0)

    def pass1(p, _):
        scan_count(lambda u, e, gi: u >= plsc.load_gather(mid, [e]))
        publish_reduce(p & 1)

        def upd(j, _):
            sl = pl.ds(j * 16, 16)
            ok = cnt_loc[sl] >= cap
            l2 = jnp.where(ok, mid[sl], lo[sl])
            h2 = jnp.where(ok, hi[sl], mid[sl])
            lo[sl] = l2
            hi[sl] = h2
            mid[sl] = l2 + ((h2 - l2) >> 1)
            return 0

        lax.fori_loop(0, 4, upd, 0)
        return 0

    lax.fori_loop(0, 25, pass1, 0)

    def sett(j, _):
        sl = pl.ds(j * 16, 16)
        tv = lo[sl]
        tt[sl] = jnp.where(tv == _LO_BITS, 0, tv)
        return 0

    lax.fori_loop(0, 4, sett, 0)

    scan_count(lambda u, e, gi: u > plsc.load_gather(tt, [e]))
    publish_reduce(1)

    def init2(j, _):
        sl = pl.ds(j * 16, 16)
        slots[sl] = cap - cnt_loc[sl]
        lo[sl] = jnp.zeros((16,), jnp.int32)
        hi[sl] = jnp.full((16,), 65536, jnp.int32)
        mid[sl] = jnp.full((16,), 32768, jnp.int32)
        return 0

    lax.fori_loop(0, 4, init2, 0)

    scan_count(lambda u, e, gi: u == plsc.load_gather(tt, [e]))
    publish_reduce(0)

    def chk(j, acc):
        sl = pl.ds(j * 16, 16)
        return jnp.maximum(acc, jnp.max(cnt_loc[sl] - slots[sl]))

    need = lax.fori_loop(0, 4, chk, jnp.int32(-(1 << 30)))

    def do_search2():
        def pass2(p, _):
            scan_count(lambda u, e, gi:
                       (u == plsc.load_gather(tt, [e]))
                       & (gi < plsc.load_gather(mid, [e])))
            publish_reduce((p + 1) & 1)

            def upd(j, _):
                sl = pl.ds(j * 16, 16)
                ok = cnt_loc[sl] <= slots[sl]
                l2 = jnp.where(ok, mid[sl], lo[sl])
                h2 = jnp.where(ok, hi[sl], mid[sl])
                lo[sl] = l2
                hi[sl] = h2
                mid[sl] = l2 + ((h2 - l2) >> 1)
                return 0

            lax.fori_loop(0, 4, upd, 0)
            return 0

        lax.fori_loop(0, 17, pass2, 0)

        def seti(j, _):
            sl = pl.ds(j * 16, 16)
            ii[sl] = lo[sl]
            return 0

        lax.fori_loop(0, 4, seti, 0)

    def skip_search2():
        full = jnp.full((16,), 65536, jnp.int32)

        def seti(j, _):
            ii[pl.ds(j * 16, 16)] = full
            return 0

        lax.fori_loop(0, 4, seti, 0)

    lax.cond(need > 0, do_search2, skip_search2)

    lbase = cid * (_TPS // 2)

    def app(j, _):
        sl = pl.ds(lbase + j * 16, 16)
        u = uv[sl]
        e = ev[sl]
        tg = plsc.load_gather(tt, [e])
        ig = plsc.load_gather(ii, [e])
        gi = base + lbase + j * 16 + lane
        keep = (u > tg) | ((u == tg) & (gi < ig))
        stage[pl.ds(j * 16, 16)] = jnp.where(keep, wv[sl], 0.0)
        return 0

    lax.fori_loop(0, _TPS // 32, app, 0)
    pltpu.sync_copy(stage, out_hbm.at[pl.ds(base + lbase, _TPS // 2)])


def _sc_select(w1, u1, e1, n, cap):
    mesh = plsc.VectorSubcoreMesh(core_axis_name="c", subcore_axis_name="s")
    k = functools.partial(
        pl.kernel,
        mesh=mesh,
        compiler_params=pltpu.CompilerParams(needs_layout_passes=False),
        out_type=jax.ShapeDtypeStruct((n,), jnp.float32),
        scratch_types=[
            pltpu.VMEM((_TPS,), jnp.float32),
            pltpu.VMEM((_TPS,), jnp.int32),
            pltpu.VMEM((_TPS,), jnp.int32),
            pltpu.VMEM((_TPS,), jnp.int32),
            pltpu.VMEM((16 * _E,), jnp.int32),
            pltpu.VMEM((_E,), jnp.int32),
            pltpu.VMEM((16 * _E,), jnp.int32),
            pltpu.VMEM((_E,), jnp.int32),
            pltpu.VMEM((_E,), jnp.int32),
            pltpu.VMEM((_E,), jnp.int32),
            pltpu.VMEM((_E,), jnp.int32),
            pltpu.VMEM((_E,), jnp.int32),
            pltpu.VMEM((_E,), jnp.int32),
            pltpu.VMEM((_TPS // 2,), jnp.float32),
            pltpu.VMEM_SHARED((2 * 16 * _E,), jnp.int32),
        ],
    )(functools.partial(_sc_body, cap=cap, n=n))
    return k(w1, u1, e1)


def kernel(hidden_states, W_gate):
    b, s, d = hidden_states.shape
    n = b * s
    e = W_gate.shape[0]
    cap = int(n * _CAPF / e)
    chunk = 2048
    grid = n // chunk
    hs2 = hidden_states.reshape(n, d)

    w1, u1, e1, psum, cnt, aux = pl.pallas_call(
        functools.partial(_phase1_body, grid=grid),
        grid=(grid,),
        in_specs=[
            pl.BlockSpec((chunk, d), lambda i: (i, 0)),
            pl.BlockSpec((e, d), lambda i: (0, 0)),
        ],
        out_specs=[
            pl.BlockSpec((1, 1, chunk), lambda i: (i, 0, 0)),
            pl.BlockSpec((1, 1, chunk), lambda i: (i, 0, 0)),
            pl.BlockSpec((1, 1, chunk), lambda i: (i, 0, 0)),
            pl.BlockSpec((e, 128), lambda i: (0, 0)),
            pl.BlockSpec((e, 128), lambda i: (0, 0)),
            pl.BlockSpec((1, 1), lambda i: (0, 0)),
        ],
        out_shape=[
            jax.ShapeDtypeStruct((grid, 1, chunk), jnp.float32),
            jax.ShapeDtypeStruct((grid, 1, chunk), jnp.int32),
            jax.ShapeDtypeStruct((grid, 1, chunk), jnp.int32),
            jax.ShapeDtypeStruct((e, 128), jnp.float32),
            jax.ShapeDtypeStruct((e, 128), jnp.float32),
            jax.ShapeDtypeStruct((1, 1), jnp.float32),
        ],
    )(hs2, W_gate)

    wk = _sc_select(w1.reshape(n), u1.reshape(n), e1.reshape(n), n, cap)
    e1 = e1.reshape(n, 1)

    return (wk.reshape(n, 1), e1, cnt[:, 0], aux[0, 0])


# speedup vs baseline: 2.2822x; 1.2286x over previous
import functools

import jax
import jax.numpy as jnp
from jax import lax
from jax.experimental import pallas as pl
from jax.experimental.pallas import tpu as pltpu
from jax.experimental.pallas import tpu_sc as plsc

_E = 64
_CAPF = 1.25
_TPS = 2048
_LO_BITS = 0x3C000000
_HI_BITS = 0x3F800000
_NLEV = 7


def _phase1_body(hs_ref, wt_ref, w_ref, u_ref, e_ref, psum_ref, cnt_ref,
                 aux_ref, *, grid):
    x = hs_ref[...]
    wg = wt_ref[...]
    logits = lax.dot_general(wg, x, (((1,), (1,)), ((), ())),
                             preferred_element_type=jnp.float32)
    m = jnp.max(logits, axis=0, keepdims=True)
    ex = jnp.exp(logits - m)
    s = jnp.sum(ex, axis=0, keepdims=True)
    wmax = 1.0 / s
    e, c = logits.shape
    iota_e = lax.broadcasted_iota(jnp.int32, (e, c), 0)
    eidx = jnp.min(jnp.where(logits == m, iota_e, e), axis=0, keepdims=True)
    w_ref[...] = wmax.reshape(1, 1, c)
    u_ref[...] = lax.bitcast_convert_type(wmax, jnp.int32).reshape(1, 1, c)
    e_ref[...] = eidx.reshape(1, 1, c)
    probs = ex * wmax
    psum_part = jnp.sum(probs, axis=1, keepdims=True)
    onehot = (iota_e == eidx).astype(jnp.float32)
    cnt_part = jnp.sum(onehot, axis=1, keepdims=True)

    @pl.when(pl.program_id(0) == 0)
    def _init():
        psum_ref[...] = jnp.zeros_like(psum_ref)
        cnt_ref[...] = jnp.zeros_like(cnt_ref)

    psum_ref[...] += jnp.broadcast_to(psum_part, psum_ref.shape)
    cnt_ref[...] += jnp.broadcast_to(cnt_part, cnt_ref.shape)

    n = c * grid

    @pl.when(pl.program_id(0) == grid - 1)
    def _aux():
        prod = jnp.sum(cnt_ref[:, 0:1] * psum_ref[:, 0:1], axis=0,
                       keepdims=True)
        aux_ref[...] = prod * (_E / (n * float(n)))


def _sc_body(w_hbm, u_hbm, e_hbm, out_hbm,
             wv, ev, uv, sidx, sidx2, hist_tab, hloc, hred, dloc, rowbuf,
             cnt_tab, cnt_loc, slab_loc, lo, hi, mid, tt, ii, slots, stage,
             smem, hist_slab, dec_slab, slab, *, cap, n):
    cid = lax.axis_index("c")
    sid = lax.axis_index("s")
    base = sid * _TPS
    nv = _TPS // 16
    pltpu.sync_copy(w_hbm.at[pl.ds(base, _TPS)], wv)
    pltpu.sync_copy(u_hbm.at[pl.ds(base, _TPS)], uv)
    pltpu.sync_copy(e_hbm.at[pl.ds(base, _TPS)], ev)
    lane = lax.iota(jnp.int32, 16)
    ones = jnp.ones((16,), jnp.int32)
    zero16 = jnp.zeros((16,), jnp.int32)

    def prep(j, _):
        sl = pl.ds(j * 16, 16)
        sidx[sl] = lane * _E + ev[sl]
        sidx2[sl] = lane * 1024 + ev[sl] * 16
        return 0

    lax.fori_loop(0, nv, prep, 0)

    def init1(j, _):
        lo[pl.ds(j * 16, 16)] = jnp.full((16,), _LO_BITS, jnp.int32)
        return 0

    lax.fori_loop(0, 4, init1, 0)

    for lev in range(_NLEV):
        shift = 4 * (_NLEV - 1 - lev)
        rng = 1 << (shift + 4)
        par = lev & 1
        last = lev == _NLEV - 1

        def zz(j, _):
            for k in range(4):
                hist_tab[pl.ds(j * 64 + k * 16, 16)] = zero16
            return 0

        lax.fori_loop(0, 1024 // 4, zz, 0)

        def scan(j, _, shift=shift, rng=rng):
            for k in range(4):
                sl = pl.ds(j * 64 + k * 16, 16)
                u = uv[sl]
                log = plsc.load_gather(lo, [ev[sl]])
                act = (u >= log) & (u < log + rng)
                d = ((u - log) >> shift) & 15
                plsc.addupdate_scatter(hist_tab, [sidx2[sl] + d], ones,
                                       mask=act)
            return 0

        lax.fori_loop(0, nv // 4, scan, 0)

        def lred(j, _):
            acc = jnp.zeros((16,), jnp.int32)
            for l in range(16):
                acc = acc + hist_tab[pl.ds(l * 1024 + j * 16, 16)]
            hloc[pl.ds(j * 16, 16)] = acc
            return 0

        lax.fori_loop(0, _E, lred, 0)

        pltpu.sync_copy(hloc, hist_slab.at[par, sid])
        plsc.subcore_barrier()

        @pl.when(sid < 8)
        def _decide(lev=lev, shift=shift, last=last, par=par):
            coff = jnp.minimum(sid, 7) * 128
            pltpu.sync_copy(hist_slab.at[par, :, pl.ds(coff, 128)], hred)
            row = zero16
            row2 = zero16
            nacc = jnp.int32(0)
            for q in range(8):
                h = jnp.zeros((16,), jnp.int32)
                for r in range(16):
                    h = h + hred[r, pl.ds(q * 16, 16)]
                if lev == 0:
                    tot = jnp.sum(h)
                    kall = (tot < cap).astype(jnp.int32)
                    smem[16 + q] = kall
                    rem = jnp.int32(cap)
                    bold = jnp.int32(0)
                else:
                    kall = smem[16 + q]
                    rem = smem[q]
                    bold = smem[8 + q]
                sfx = lax.rev(jnp.cumsum(lax.rev(h, (0,))), (0,))
                g = sfx - h
                cond = (g < rem) & ((g + h) >= rem)
                dstar = jnp.max(jnp.where(cond, lane, -1))
                gsel = jnp.max(jnp.where(cond, g, 0))
                hsel = jnp.max(jnp.where(cond, h, 0))
                knz = kall == 1
                dse = jnp.where(knz, 0, jnp.maximum(dstar, 0))
                bnew = bold + dse * (1 << shift)
                rnew = jnp.where(knz, rem, rem - gsel)
                smem[q] = rnew
                smem[8 + q] = bnew
                if last:
                    tq = jnp.where(knz, 0, _LO_BITS + bnew)
                    row = jnp.where(lane == q, tq, row)
                    row = jnp.where(lane == 8 + q, rnew, row)
                    nacc = jnp.maximum(nacc, jnp.where(knz, 0, hsel - rnew))
                else:
                    row = jnp.where(lane == q, _LO_BITS + bnew, row)
            rowbuf[...] = row
            pltpu.sync_copy(rowbuf, dec_slab.at[par, sid])
            if last:
                row2 = jnp.where(lane == 0, nacc, row2)
                rowbuf[...] = row2
                pltpu.sync_copy(rowbuf, dec_slab.at[par, sid + 8])

        plsc.subcore_barrier()

        pltpu.sync_copy(dec_slab.at[par], dloc)
        if not last:
            def rebuild(j, _):
                e16 = j * 16 + lane
                lo[pl.ds(j * 16, 16)] = plsc.load_gather(
                    dloc, [e16 >> 3, e16 & 7])
                return 0

            lax.fori_loop(0, 4, rebuild, 0)

    def unpack(j, _):
        e16 = j * 16 + lane
        tt[pl.ds(j * 16, 16)] = plsc.load_gather(dloc, [e16 >> 3, e16 & 7])
        slots[pl.ds(j * 16, 16)] = plsc.load_gather(
            dloc, [e16 >> 3, 8 + (e16 & 7)])
        return 0

    lax.fori_loop(0, 4, unpack, 0)
    need = jnp.max(plsc.load_gather(dloc, [8 + (lane & 7),
                                           jnp.zeros((16,), jnp.int32)]))

    def scan_count(pred):
        def z(j, _):
            for k in range(4):
                cnt_tab[pl.ds(j * 64 + k * 16, 16)] = zero16
            return 0

        lax.fori_loop(0, _E // 4, z, 0)

        def sc(j, _):
            for k in range(4):
                sl = pl.ds(j * 64 + k * 16, 16)
                bit = pred(uv[sl], ev[sl], base + j * 64 + k * 16 + lane)
                plsc.addupdate_scatter(cnt_tab, [sidx[sl]], ones, mask=bit)
            return 0

        lax.fori_loop(0, nv // 4, sc, 0)

        def red(j, _):
            acc = jnp.zeros((16,), jnp.int32)
            for l in range(16):
                acc = acc + cnt_tab[pl.ds(l * _E + j * 16, 16)]
            cnt_loc[pl.ds(j * 16, 16)] = acc
            return 0

        lax.fori_loop(0, 4, red, 0)

    def publish_reduce(par):
        pltpu.sync_copy(cnt_loc, slab.at[pl.ds((par * 16 + sid) * _E, _E)])
        plsc.subcore_barrier()
        pltpu.sync_copy(slab.at[pl.ds(par * 16 * _E, 16 * _E)], slab_loc)

        def red2(j, _):
            acc = jnp.zeros((16,), jnp.int32)
            for r in range(16):
                acc = acc + slab_loc[pl.ds(r * _E + j * 16, 16)]
            cnt_loc[pl.ds(j * 16, 16)] = acc
            return 0

        lax.fori_loop(0, 4, red2, 0)

    def do_search2():
        def init2(j, _):
            sl = pl.ds(j * 16, 16)
            lo[sl] = jnp.zeros((16,), jnp.int32)
            hi[sl] = jnp.full((16,), 65536, jnp.int32)
            mid[sl] = jnp.full((16,), 32768, jnp.int32)
            return 0

        lax.fori_loop(0, 4, init2, 0)

        def pass2(p, _):
            scan_count(lambda u, e, gi:
                       (u == plsc.load_gather(tt, [e]))
                       & (gi < plsc.load_gather(mid, [e])))
            publish_reduce((p + 1) & 1)

            def upd(j, _):
                sl = pl.ds(j * 16, 16)
                ok = cnt_loc[sl] <= slots[sl]
                l2 = jnp.where(ok, mid[sl], lo[sl])
                h2 = jnp.where(ok, hi[sl], mid[sl])
                lo[sl] = l2
                hi[sl] = h2
                mid[sl] = l2 + ((h2 - l2) >> 1)
                return 0

            lax.fori_loop(0, 4, upd, 0)
            return 0

        lax.fori_loop(0, 17, pass2, 0)

        def seti(j, _):
            sl = pl.ds(j * 16, 16)
            ii[sl] = lo[sl]
            return 0

        lax.fori_loop(0, 4, seti, 0)

    def skip_search2():
        full = jnp.full((16,), 65536, jnp.int32)

        def seti(j, _):
            ii[pl.ds(j * 16, 16)] = full
            return 0

        lax.fori_loop(0, 4, seti, 0)

    lax.cond(need > 0, do_search2, skip_search2)

    lbase = cid * (_TPS // 2)

    def app(j, _):
        sl = pl.ds(lbase + j * 16, 16)
        u = uv[sl]
        e = ev[sl]
        tg = plsc.load_gather(tt, [e])
        ig = plsc.load_gather(ii, [e])
        gi = base + lbase + j * 16 + lane
        keep = (u > tg) | ((u == tg) & (gi < ig))
        stage[pl.ds(j * 16, 16)] = jnp.where(keep, wv[sl], 0.0)
        return 0

    lax.fori_loop(0, _TPS // 32, app, 0)
    pltpu.sync_copy(stage, out_hbm.at[pl.ds(base + lbase, _TPS // 2)])


def _sc_select(w1, u1, e1, n, cap):
    mesh = plsc.VectorSubcoreMesh(core_axis_name="c", subcore_axis_name="s")
    k = functools.partial(
        pl.kernel,
        mesh=mesh,
        compiler_params=pltpu.CompilerParams(needs_layout_passes=False),
        out_type=jax.ShapeDtypeStruct((n,), jnp.float32),
        scratch_types=[
            pltpu.VMEM((_TPS,), jnp.float32),
            pltpu.VMEM((_TPS,), jnp.int32),
            pltpu.VMEM((_TPS,), jnp.int32),
            pltpu.VMEM((_TPS,), jnp.int32),
            pltpu.VMEM((_TPS,), jnp.int32),
            pltpu.VMEM((16 * 1024,), jnp.int32),
            pltpu.VMEM((1024,), jnp.int32),
            pltpu.VMEM((16, 128), jnp.int32),
            pltpu.VMEM((16, 16), jnp.int32),
            pltpu.VMEM((16,), jnp.int32),
            pltpu.VMEM((16 * _E,), jnp.int32),
            pltpu.VMEM((_E,), jnp.int32),
            pltpu.VMEM((16 * _E,), jnp.int32),
            pltpu.VMEM((_E,), jnp.int32),
            pltpu.VMEM((_E,), jnp.int32),
            pltpu.VMEM((_E,), jnp.int32),
            pltpu.VMEM((_E,), jnp.int32),
            pltpu.VMEM((_E,), jnp.int32),
            pltpu.VMEM((_E,), jnp.int32),
            pltpu.VMEM((_TPS // 2,), jnp.float32),
            pltpu.SMEM((32,), jnp.int32),
            pltpu.VMEM_SHARED((2, 16, 1024), jnp.int32),
            pltpu.VMEM_SHARED((2, 16, 16), jnp.int32),
            pltpu.VMEM_SHARED((2 * 16 * _E,), jnp.int32),
        ],
    )(functools.partial(_sc_body, cap=cap, n=n))
    return k(w1, u1, e1)


def kernel(hidden_states, W_gate):
    b, s, d = hidden_states.shape
    n = b * s
    e = W_gate.shape[0]
    cap = int(n * _CAPF / e)
    chunk = 2048
    grid = n // chunk
    hs2 = hidden_states.reshape(n, d)

    w1, u1, e1, psum, cnt, aux = pl.pallas_call(
        functools.partial(_phase1_body, grid=grid),
        grid=(grid,),
        in_specs=[
            pl.BlockSpec((chunk, d), lambda i: (i, 0)),
            pl.BlockSpec((e, d), lambda i: (0, 0)),
        ],
        out_specs=[
            pl.BlockSpec((1, 1, chunk), lambda i: (i, 0, 0)),
            pl.BlockSpec((1, 1, chunk), lambda i: (i, 0, 0)),
            pl.BlockSpec((1, 1, chunk), lambda i: (i, 0, 0)),
            pl.BlockSpec((e, 128), lambda i: (0, 0)),
            pl.BlockSpec((e, 128), lambda i: (0, 0)),
            pl.BlockSpec((1, 1), lambda i: (0, 0)),
        ],
        out_shape=[
            jax.ShapeDtypeStruct((grid, 1, chunk), jnp.float32),
            jax.ShapeDtypeStruct((grid, 1, chunk), jnp.int32),
            jax.ShapeDtypeStruct((grid, 1, chunk), jnp.int32),
            jax.ShapeDtypeStruct((e, 128), jnp.float32),
            jax.ShapeDtypeStruct((e, 128), jnp.float32),
            jax.ShapeDtypeStruct((1, 1), jnp.float32),
        ],
    )(hs2, W_gate)

    wk = _sc_select(w1.reshape(n), u1.reshape(n), e1.reshape(n), n, cap)
    e1 = e1.reshape(n, 1)

    return (wk.reshape(n, 1), e1, cnt[:, 0], aux[0, 0])


# speedup vs baseline: 2.4669x; 1.0809x over previous
import functools

import jax
import jax.numpy as jnp
from jax import lax
from jax.experimental import pallas as pl
from jax.experimental.pallas import tpu as pltpu
from jax.experimental.pallas import tpu_sc as plsc

_E = 64
_CAPF = 1.25
_TPS = 2048
_LO_BITS = 0x3C000000
_HI_BITS = 0x3F800000
_NLEV = 7


def _phase1_body(hs_ref, wt_ref, w_ref, u_ref, e_ref, psum_ref, cnt_ref,
                 aux_ref, *, grid):
    x = hs_ref[...]
    wg = wt_ref[...]
    logits = lax.dot_general(wg, x, (((1,), (1,)), ((), ())),
                             preferred_element_type=jnp.float32)
    m = jnp.max(logits, axis=0, keepdims=True)
    ex = jnp.exp(logits - m)
    s = jnp.sum(ex, axis=0, keepdims=True)
    wmax = 1.0 / s
    e, c = logits.shape
    iota_e = lax.broadcasted_iota(jnp.int32, (e, c), 0)
    eidx = jnp.min(jnp.where(logits == m, iota_e, e), axis=0, keepdims=True)
    w_ref[...] = wmax.reshape(1, 1, c)
    u_ref[...] = lax.bitcast_convert_type(wmax, jnp.int32).reshape(1, 1, c)
    e_ref[...] = eidx.reshape(1, 1, c)
    probs = ex * wmax
    psum_part = jnp.sum(probs, axis=1, keepdims=True)
    onehot = (iota_e == eidx).astype(jnp.float32)
    cnt_part = jnp.sum(onehot, axis=1, keepdims=True)

    @pl.when(pl.program_id(0) == 0)
    def _init():
        psum_ref[...] = jnp.zeros_like(psum_ref)
        cnt_ref[...] = jnp.zeros_like(cnt_ref)

    psum_ref[...] += jnp.broadcast_to(psum_part, psum_ref.shape)
    cnt_ref[...] += jnp.broadcast_to(cnt_part, cnt_ref.shape)

    n = c * grid

    @pl.when(pl.program_id(0) == grid - 1)
    def _aux():
        prod = jnp.sum(cnt_ref[:, 0:1] * psum_ref[:, 0:1], axis=0,
                       keepdims=True)
        aux_ref[...] = prod * (_E / (n * float(n)))


def _sc_body(w_hbm, u_hbm, e_hbm, out_hbm,
             wv, ev, uv, sidx, sidx2, hist_tab, hloc, hred, dloc, rowbuf,
             cnt_tab, cnt_loc, slab_loc, lo, hi, mid, tt, ii, slots, stage,
             smem, hist_slab, dec_slab, slab, *, cap, n):
    cid = lax.axis_index("c")
    sid = lax.axis_index("s")
    base = sid * _TPS
    nv = _TPS // 16
    pltpu.sync_copy(w_hbm.at[pl.ds(base, _TPS)], wv)
    pltpu.sync_copy(u_hbm.at[pl.ds(base, _TPS)], uv)
    pltpu.sync_copy(e_hbm.at[pl.ds(base, _TPS)], ev)
    lane = lax.iota(jnp.int32, 16)
    ones = jnp.ones((16,), jnp.int32)
    zero16 = jnp.zeros((16,), jnp.int32)

    def prep(j, _):
        sl = pl.ds(j * 16, 16)
        sidx2[sl] = lane * 1024 + ev[sl] * 16
        return 0

    lax.fori_loop(0, nv, prep, 0)

    def init1(j, _):
        lo[pl.ds(j * 16, 16)] = jnp.full((16,), _LO_BITS, jnp.int32)
        return 0

    lax.fori_loop(0, 4, init1, 0)

    def zz(j, _):
        for k in range(4):
            hist_tab[pl.ds(j * 64 + k * 16, 16)] = zero16
        return 0

    lax.fori_loop(0, 1024 // 4, zz, 0)

    for lev in range(_NLEV):
        shift = 4 * (_NLEV - 1 - lev)
        rng = 1 << (shift + 4)
        par = lev & 1
        last = lev == _NLEV - 1

        if lev == 0:
            def scan0(j, _):
                for k in range(4):
                    sl = pl.ds(j * 64 + k * 16, 16)
                    d = (uv[sl] - _LO_BITS) >> 24
                    plsc.addupdate_scatter(hist_tab, [sidx2[sl] + d], ones)
                return 0

            lax.fori_loop(0, nv // 4, scan0, 0)
        else:
            def scan(j, _, shift=shift, rng=rng):
                for k in range(4):
                    sl = pl.ds(j * 64 + k * 16, 16)
                    u = uv[sl]
                    log = plsc.load_gather(lo, [ev[sl]])
                    act = (u >= log) & (u < log + rng)
                    d = ((u - log) >> shift) & 15
                    plsc.addupdate_scatter(hist_tab, [sidx2[sl] + d], ones,
                                           mask=act)
                return 0

            lax.fori_loop(0, nv // 4, scan, 0)

        def lred(j, _):
            acc = jnp.zeros((16,), jnp.int32)
            for l in range(16):
                sl = pl.ds(l * 1024 + j * 16, 16)
                acc = acc + hist_tab[sl]
                hist_tab[sl] = zero16
            hloc[pl.ds(j * 16, 16)] = acc
            return 0

        lax.fori_loop(0, _E, lred, 0)

        pltpu.sync_copy(hloc, hist_slab.at[par, sid])
        plsc.subcore_barrier()

        @pl.when(sid < 8)
        def _decide(lev=lev, shift=shift, last=last, par=par):
            coff = jnp.minimum(sid, 7) * 128
            pltpu.sync_copy(hist_slab.at[par, :, pl.ds(coff, 128)], hred)
            row = zero16
            row2 = zero16
            nacc = jnp.int32(0)
            for q in range(8):
                h = jnp.zeros((16,), jnp.int32)
                for r in range(16):
                    h = h + hred[r, pl.ds(q * 16, 16)]
                if lev == 0:
                    tot = jnp.sum(h)
                    kall = (tot < cap).astype(jnp.int32)
                    smem[16 + q] = kall
                    rem = jnp.int32(cap)
                    bold = jnp.int32(0)
                else:
                    kall = smem[16 + q]
                    rem = smem[q]
                    bold = smem[8 + q]
                sfx = lax.rev(jnp.cumsum(lax.rev(h, (0,))), (0,))
                g = sfx - h
                cond = (g < rem) & ((g + h) >= rem)
                dstar = jnp.max(jnp.where(cond, lane, -1))
                gsel = jnp.max(jnp.where(cond, g, 0))
                hsel = jnp.max(jnp.where(cond, h, 0))
                knz = kall == 1
                dse = jnp.where(knz, 0, jnp.maximum(dstar, 0))
                bnew = bold + dse * (1 << shift)
                rnew = jnp.where(knz, rem, rem - gsel)
                smem[q] = rnew
                smem[8 + q] = bnew
                if last:
                    tq = jnp.where(knz, 0, _LO_BITS + bnew)
                    row = jnp.where(lane == q, tq, row)
                    row = jnp.where(lane == 8 + q, rnew, row)
                    nacc = jnp.maximum(nacc, jnp.where(knz, 0, hsel - rnew))
                else:
                    row = jnp.where(lane == q, _LO_BITS + bnew, row)
            rowbuf[...] = row
            pltpu.sync_copy(rowbuf, dec_slab.at[par, sid])
            if last:
                row2 = jnp.where(lane == 0, nacc, row2)
                rowbuf[...] = row2
                pltpu.sync_copy(rowbuf, dec_slab.at[par, sid + 8])

        plsc.subcore_barrier()

        pltpu.sync_copy(dec_slab.at[par], dloc)
        if not last:
            def rebuild(j, _):
                e16 = j * 16 + lane
                lo[pl.ds(j * 16, 16)] = plsc.load_gather(
                    dloc, [e16 >> 3, e16 & 7])
                return 0

            lax.fori_loop(0, 4, rebuild, 0)

    def unpack(j, _):
        e16 = j * 16 + lane
        tt[pl.ds(j * 16, 16)] = plsc.load_gather(dloc, [e16 >> 3, e16 & 7])
        slots[pl.ds(j * 16, 16)] = plsc.load_gather(
            dloc, [e16 >> 3, 8 + (e16 & 7)])
        return 0

    lax.fori_loop(0, 4, unpack, 0)
    need = jnp.max(plsc.load_gather(dloc, [8 + (lane & 7),
                                           jnp.zeros((16,), jnp.int32)]))

    def scan_count(pred):
        def z(j, _):
            for k in range(4):
                cnt_tab[pl.ds(j * 64 + k * 16, 16)] = zero16
            return 0

        lax.fori_loop(0, _E // 4, z, 0)

        def sc(j, _):
            for k in range(4):
                sl = pl.ds(j * 64 + k * 16, 16)
                bit = pred(uv[sl], ev[sl], base + j * 64 + k * 16 + lane)
                plsc.addupdate_scatter(cnt_tab, [sidx[sl]], ones, mask=bit)
            return 0

        lax.fori_loop(0, nv // 4, sc, 0)

        def red(j, _):
            acc = jnp.zeros((16,), jnp.int32)
            for l in range(16):
                acc = acc + cnt_tab[pl.ds(l * _E + j * 16, 16)]
            cnt_loc[pl.ds(j * 16, 16)] = acc
            return 0

        lax.fori_loop(0, 4, red, 0)

    def publish_reduce(par):
        pltpu.sync_copy(cnt_loc, slab.at[pl.ds((par * 16 + sid) * _E, _E)])
        plsc.subcore_barrier()
        pltpu.sync_copy(slab.at[pl.ds(par * 16 * _E, 16 * _E)], slab_loc)

        def red2(j, _):
            acc = jnp.zeros((16,), jnp.int32)
            for r in range(16):
                acc = acc + slab_loc[pl.ds(r * _E + j * 16, 16)]
            cnt_loc[pl.ds(j * 16, 16)] = acc
            return 0

        lax.fori_loop(0, 4, red2, 0)

    def do_search2():
        def prep2(j, _):
            sl = pl.ds(j * 16, 16)
            sidx[sl] = lane * _E + ev[sl]
            return 0

        lax.fori_loop(0, nv, prep2, 0)

        def init2(j, _):
            sl = pl.ds(j * 16, 16)
            lo[sl] = jnp.zeros((16,), jnp.int32)
            hi[sl] = jnp.full((16,), 65536, jnp.int32)
            mid[sl] = jnp.full((16,), 32768, jnp.int32)
            return 0

        lax.fori_loop(0, 4, init2, 0)

        def pass2(p, _):
            scan_count(lambda u, e, gi:
                       (u == plsc.load_gather(tt, [e]))
                       & (gi < plsc.load_gather(mid, [e])))
            publish_reduce((p + 1) & 1)

            def upd(j, _):
                sl = pl.ds(j * 16, 16)
                ok = cnt_loc[sl] <= slots[sl]
                l2 = jnp.where(ok, mid[sl], lo[sl])
                h2 = jnp.where(ok, hi[sl], mid[sl])
                lo[sl] = l2
                hi[sl] = h2
                mid[sl] = l2 + ((h2 - l2) >> 1)
                return 0

            lax.fori_loop(0, 4, upd, 0)
            return 0

        lax.fori_loop(0, 17, pass2, 0)

        def seti(j, _):
            sl = pl.ds(j * 16, 16)
            ii[sl] = lo[sl]
            return 0

        lax.fori_loop(0, 4, seti, 0)

    def skip_search2():
        full = jnp.full((16,), 65536, jnp.int32)

        def seti(j, _):
            ii[pl.ds(j * 16, 16)] = full
            return 0

        lax.fori_loop(0, 4, seti, 0)

    lax.cond(need > 0, do_search2, skip_search2)

    lbase = cid * (_TPS // 2)

    def app(j, _):
        sl = pl.ds(lbase + j * 16, 16)
        u = uv[sl]
        e = ev[sl]
        tg = plsc.load_gather(tt, [e])
        ig = plsc.load_gather(ii, [e])
        gi = base + lbase + j * 16 + lane
        keep = (u > tg) | ((u == tg) & (gi < ig))
        stage[pl.ds(j * 16, 16)] = jnp.where(keep, wv[sl], 0.0)
        return 0

    lax.fori_loop(0, _TPS // 32, app, 0)
    pltpu.sync_copy(stage, out_hbm.at[pl.ds(base + lbase, _TPS // 2)])


def _sc_select(w1, u1, e1, n, cap):
    mesh = plsc.VectorSubcoreMesh(core_axis_name="c", subcore_axis_name="s")
    k = functools.partial(
        pl.kernel,
        mesh=mesh,
        compiler_params=pltpu.CompilerParams(needs_layout_passes=False),
        out_type=jax.ShapeDtypeStruct((n,), jnp.float32),
        scratch_types=[
            pltpu.VMEM((_TPS,), jnp.float32),
            pltpu.VMEM((_TPS,), jnp.int32),
            pltpu.VMEM((_TPS,), jnp.int32),
            pltpu.VMEM((_TPS,), jnp.int32),
            pltpu.VMEM((_TPS,), jnp.int32),
            pltpu.VMEM((16 * 1024,), jnp.int32),
            pltpu.VMEM((1024,), jnp.int32),
            pltpu.VMEM((16, 128), jnp.int32),
            pltpu.VMEM((16, 16), jnp.int32),
            pltpu.VMEM((16,), jnp.int32),
            pltpu.VMEM((16 * _E,), jnp.int32),
            pltpu.VMEM((_E,), jnp.int32),
            pltpu.VMEM((16 * _E,), jnp.int32),
            pltpu.VMEM((_E,), jnp.int32),
            pltpu.VMEM((_E,), jnp.int32),
            pltpu.VMEM((_E,), jnp.int32),
            pltpu.VMEM((_E,), jnp.int32),
            pltpu.VMEM((_E,), jnp.int32),
            pltpu.VMEM((_E,), jnp.int32),
            pltpu.VMEM((_TPS // 2,), jnp.float32),
            pltpu.SMEM((32,), jnp.int32),
            pltpu.VMEM_SHARED((2, 16, 1024), jnp.int32),
            pltpu.VMEM_SHARED((2, 16, 16), jnp.int32),
            pltpu.VMEM_SHARED((2 * 16 * _E,), jnp.int32),
        ],
    )(functools.partial(_sc_body, cap=cap, n=n))
    return k(w1, u1, e1)


def kernel(hidden_states, W_gate):
    b, s, d = hidden_states.shape
    n = b * s
    e = W_gate.shape[0]
    cap = int(n * _CAPF / e)
    chunk = 2048
    grid = n // chunk
    hs2 = hidden_states.reshape(n, d)

    w1, u1, e1, psum, cnt, aux = pl.pallas_call(
        functools.partial(_phase1_body, grid=grid),
        grid=(grid,),
        in_specs=[
            pl.BlockSpec((chunk, d), lambda i: (i, 0)),
            pl.BlockSpec((e, d), lambda i: (0, 0)),
        ],
        out_specs=[
            pl.BlockSpec((1, 1, chunk), lambda i: (i, 0, 0)),
            pl.BlockSpec((1, 1, chunk), lambda i: (i, 0, 0)),
            pl.BlockSpec((1, 1, chunk), lambda i: (i, 0, 0)),
            pl.BlockSpec((e, 128), lambda i: (0, 0)),
            pl.BlockSpec((e, 128), lambda i: (0, 0)),
            pl.BlockSpec((1, 1), lambda i: (0, 0)),
        ],
        out_shape=[
            jax.ShapeDtypeStruct((grid, 1, chunk), jnp.float32),
            jax.ShapeDtypeStruct((grid, 1, chunk), jnp.int32),
            jax.ShapeDtypeStruct((grid, 1, chunk), jnp.int32),
            jax.ShapeDtypeStruct((e, 128), jnp.float32),
            jax.ShapeDtypeStruct((e, 128), jnp.float32),
            jax.ShapeDtypeStruct((1, 1), jnp.float32),
        ],
    )(hs2, W_gate)

    wk = _sc_select(w1.reshape(n), u1.reshape(n), e1.reshape(n), n, cap)
    e1 = e1.reshape(n, 1)

    return (wk.reshape(n, 1), e1, cnt[:, 0], aux[0, 0])


# speedup vs baseline: 2.5409x; 1.0300x over previous
import functools

import jax
import jax.numpy as jnp
from jax import lax
from jax.experimental import pallas as pl
from jax.experimental.pallas import tpu as pltpu
from jax.experimental.pallas import tpu_sc as plsc

_E = 64
_CAPF = 1.25
_TPS = 2048
_LO_BITS = 0x3C000000
_HI_BITS = 0x3F800000
_NLEV = 7


def _phase1_body(hs_ref, wt_ref, w_ref, u_ref, e_ref, psum_ref, cnt_ref,
                 aux_ref, *, grid):
    x = hs_ref[...]
    wg = wt_ref[...]
    logits = lax.dot_general(wg, x, (((1,), (1,)), ((), ())),
                             preferred_element_type=jnp.float32)
    m = jnp.max(logits, axis=0, keepdims=True)
    ex = jnp.exp(logits - m)
    s = jnp.sum(ex, axis=0, keepdims=True)
    wmax = 1.0 / s
    e, c = logits.shape
    iota_e = lax.broadcasted_iota(jnp.int32, (e, c), 0)
    eidx = jnp.min(jnp.where(logits == m, iota_e, e), axis=0, keepdims=True)
    w_ref[...] = wmax.reshape(1, 1, c)
    u_ref[...] = lax.bitcast_convert_type(wmax, jnp.int32).reshape(1, 1, c)
    e_ref[...] = eidx.reshape(1, 1, c)
    probs = ex * wmax
    psum_part = jnp.sum(probs, axis=1, keepdims=True)
    onehot = (iota_e == eidx).astype(jnp.float32)
    cnt_part = jnp.sum(onehot, axis=1, keepdims=True)

    @pl.when(pl.program_id(0) == 0)
    def _init():
        psum_ref[...] = jnp.zeros_like(psum_ref)
        cnt_ref[...] = jnp.zeros_like(cnt_ref)

    psum_ref[...] += jnp.broadcast_to(psum_part, psum_ref.shape)
    cnt_ref[...] += jnp.broadcast_to(cnt_part, cnt_ref.shape)

    n = c * grid

    @pl.when(pl.program_id(0) == grid - 1)
    def _aux():
        prod = jnp.sum(cnt_ref[:, 0:1] * psum_ref[:, 0:1], axis=0,
                       keepdims=True)
        aux_ref[...] = prod * (_E / (n * float(n)))


def _sc_body(w_hbm, u_hbm, e_hbm, out_hbm,
             wv, ev, uv, sidx, sidx2, hist_tab, hloc, hred, dloc, rowbuf,
             cnt_tab, cnt_loc, slab_loc, lo, hi, mid, tt, ii, slots, stage,
             smem, hist_slab, dec_slab, slab, *, cap, n):
    cid = lax.axis_index("c")
    sid = lax.axis_index("s")
    base = sid * _TPS
    nv = _TPS // 16
    pltpu.sync_copy(w_hbm.at[pl.ds(base, _TPS)], wv)
    pltpu.sync_copy(u_hbm.at[pl.ds(base, _TPS)], uv)
    pltpu.sync_copy(e_hbm.at[pl.ds(base, _TPS)], ev)
    lane = lax.iota(jnp.int32, 16)
    ones = jnp.ones((16,), jnp.int32)
    zero16 = jnp.zeros((16,), jnp.int32)

    def prep(j, _):
        sl = pl.ds(j * 16, 16)
        sidx2[sl] = lane * 1024 + ev[sl] * 16
        return 0

    lax.fori_loop(0, nv, prep, 0)

    def init1(j, _):
        lo[pl.ds(j * 16, 16)] = jnp.full((16,), _LO_BITS, jnp.int32)
        return 0

    lax.fori_loop(0, 4, init1, 0)

    def zz(j, _):
        for k in range(4):
            hist_tab[pl.ds(j * 64 + k * 16, 16)] = zero16
        return 0

    lax.fori_loop(0, 1024 // 4, zz, 0)

    for lev in range(_NLEV):
        shift = 4 * (_NLEV - 1 - lev)
        rng = 1 << (shift + 4)
        par = lev & 1
        last = lev == _NLEV - 1

        if lev == 0:
            def scan0(j, _):
                for k in range(4):
                    sl = pl.ds(j * 64 + k * 16, 16)
                    d = (uv[sl] - _LO_BITS) >> 24
                    plsc.addupdate_scatter(hist_tab, [sidx2[sl] + d], ones)
                return 0

            lax.fori_loop(0, nv // 4, scan0, 0)
        else:
            def scan(j, _, shift=shift, rng=rng):
                for k in range(4):
                    sl = pl.ds(j * 64 + k * 16, 16)
                    u = uv[sl]
                    log = plsc.load_gather(lo, [ev[sl]])
                    act = (u >= log) & (u < log + rng)
                    d = ((u - log) >> shift) & 15
                    plsc.addupdate_scatter(hist_tab, [sidx2[sl] + d], ones,
                                           mask=act)
                return 0

            lax.fori_loop(0, nv // 4, scan, 0)

        def lred(j, _):
            acc = jnp.zeros((16,), jnp.int32)
            for l in range(16):
                sl = pl.ds(l * 1024 + j * 16, 16)
                acc = acc + hist_tab[sl]
                hist_tab[sl] = zero16
            hloc[pl.ds(j * 16, 16)] = acc
            return 0

        lax.fori_loop(0, _E, lred, 0)

        pltpu.sync_copy(hloc, hist_slab.at[par, sid])
        plsc.subcore_barrier()

        @pl.when(sid < 8)
        def _decide(lev=lev, shift=shift, last=last, par=par):
            coff = jnp.minimum(sid, 7) * 128
            pltpu.sync_copy(hist_slab.at[par, :, pl.ds(coff, 128)], hred)
            row = zero16
            row2 = zero16
            nacc = jnp.int32(0)
            for q in range(8):
                h = jnp.zeros((16,), jnp.int32)
                for r in range(16):
                    h = h + hred[r, pl.ds(q * 16, 16)]
                if lev == 0:
                    tot = jnp.sum(h)
                    kall = (tot < cap).astype(jnp.int32)
                    smem[16 + q] = kall
                    rem = jnp.int32(cap)
                    bold = jnp.int32(0)
                else:
                    kall = smem[16 + q]
                    rem = smem[q]
                    bold = smem[8 + q]
                sfx = lax.rev(jnp.cumsum(lax.rev(h, (0,))), (0,))
                g = sfx - h
                cond = (g < rem) & ((g + h) >= rem)
                dstar = jnp.max(jnp.where(cond, lane, -1))
                gsel = jnp.max(jnp.where(cond, g, 0))
                hsel = jnp.max(jnp.where(cond, h, 0))
                knz = kall == 1
                dse = jnp.where(knz, 0, jnp.maximum(dstar, 0))
                bnew = bold + dse * (1 << shift)
                rnew = jnp.where(knz, rem, rem - gsel)
                smem[q] = rnew
                smem[8 + q] = bnew
                if last:
                    tq = jnp.where(knz, 0, _LO_BITS + bnew)
                    row = jnp.where(lane == q, tq, row)
                    row = jnp.where(lane == 8 + q, rnew, row)
                    nacc = jnp.maximum(nacc, jnp.where(knz, 0, hsel - rnew))
                else:
                    row = jnp.where(lane == q, _LO_BITS + bnew, row)
            rowbuf[...] = row
            pltpu.sync_copy(rowbuf, dec_slab.at[par, sid])
            if last:
                row2 = jnp.where(lane == 0, nacc, row2)
                rowbuf[...] = row2
                pltpu.sync_copy(rowbuf, dec_slab.at[par, sid + 8])

        plsc.subcore_barrier()

        pltpu.sync_copy(dec_slab.at[par], dloc)
        if not last:
            def rebuild(j, _):
                e16 = j * 16 + lane
                lo[pl.ds(j * 16, 16)] = plsc.load_gather(
                    dloc, [e16 >> 3, e16 & 7])
                return 0

            lax.fori_loop(0, 4, rebuild, 0)

    def unpack(j, _):
        e16 = j * 16 + lane
        tt[pl.ds(j * 16, 16)] = plsc.load_gather(dloc, [e16 >> 3, e16 & 7])
        slots[pl.ds(j * 16, 16)] = plsc.load_gather(
            dloc, [e16 >> 3, 8 + (e16 & 7)])
        return 0

    lax.fori_loop(0, 4, unpack, 0)
    need = jnp.max(plsc.load_gather(dloc, [8 + (lane & 7),
                                           jnp.zeros((16,), jnp.int32)]))

    def scan_count(pred):
        def z(j, _):
            for k in range(4):
                cnt_tab[pl.ds(j * 64 + k * 16, 16)] = zero16
            return 0

        lax.fori_loop(0, _E // 4, z, 0)

        def sc(j, _):
            for k in range(4):
                sl = pl.ds(j * 64 + k * 16, 16)
                bit = pred(uv[sl], ev[sl], base + j * 64 + k * 16 + lane)
                plsc.addupdate_scatter(cnt_tab, [sidx[sl]], ones, mask=bit)
            return 0

        lax.fori_loop(0, nv // 4, sc, 0)

        def red(j, _):
            acc = jnp.zeros((16,), jnp.int32)
            for l in range(16):
                acc = acc + cnt_tab[pl.ds(l * _E + j * 16, 16)]
            cnt_loc[pl.ds(j * 16, 16)] = acc
            return 0

        lax.fori_loop(0, 4, red, 0)

    def publish_reduce(par):
        pltpu.sync_copy(cnt_loc, slab.at[pl.ds((par * 16 + sid) * _E, _E)])
        plsc.subcore_barrier()
        pltpu.sync_copy(slab.at[pl.ds(par * 16 * _E, 16 * _E)], slab_loc)

        def red2(j, _):
            acc = jnp.zeros((16,), jnp.int32)
            for r in range(16):
                acc = acc + slab_loc[pl.ds(r * _E + j * 16, 16)]
            cnt_loc[pl.ds(j * 16, 16)] = acc
            return 0

        lax.fori_loop(0, 4, red2, 0)

    def do_search2():
        def prep2(j, _):
            sl = pl.ds(j * 16, 16)
            sidx[sl] = lane * _E + ev[sl]
            return 0

        lax.fori_loop(0, nv, prep2, 0)

        def init2(j, _):
            sl = pl.ds(j * 16, 16)
            lo[sl] = jnp.zeros((16,), jnp.int32)
            hi[sl] = jnp.full((16,), 65536, jnp.int32)
            mid[sl] = jnp.full((16,), 32768, jnp.int32)
            return 0

        lax.fori_loop(0, 4, init2, 0)

        def pass2(p, _):
            scan_count(lambda u, e, gi:
                       (u == plsc.load_gather(tt, [e]))
                       & (gi < plsc.load_gather(mid, [e])))
            publish_reduce((p + 1) & 1)

            def upd(j, _):
                sl = pl.ds(j * 16, 16)
                ok = cnt_loc[sl] <= slots[sl]
                l2 = jnp.where(ok, mid[sl], lo[sl])
                h2 = jnp.where(ok, hi[sl], mid[sl])
                lo[sl] = l2
                hi[sl] = h2
                mid[sl] = l2 + ((h2 - l2) >> 1)
                return 0

            lax.fori_loop(0, 4, upd, 0)
            return 0

        lax.fori_loop(0, 17, pass2, 0)

        def seti(j, _):
            sl = pl.ds(j * 16, 16)
            ii[sl] = lo[sl]
            return 0

        lax.fori_loop(0, 4, seti, 0)

    def skip_search2():
        full = jnp.full((16,), 65536, jnp.int32)

        def seti(j, _):
            ii[pl.ds(j * 16, 16)] = full
            return 0

        lax.fori_loop(0, 4, seti, 0)

    lax.cond(need > 0, do_search2, skip_search2)

    lbase = cid * (_TPS // 2)

    def app(j, _):
        sl = pl.ds(lbase + j * 16, 16)
        u = uv[sl]
        e = ev[sl]
        tg = plsc.load_gather(tt, [e])
        ig = plsc.load_gather(ii, [e])
        gi = base + lbase + j * 16 + lane
        keep = (u > tg) | ((u == tg) & (gi < ig))
        stage[pl.ds(j * 16, 16)] = jnp.where(keep, wv[sl], 0.0)
        return 0

    lax.fori_loop(0, _TPS // 32, app, 0)
    pltpu.sync_copy(stage, out_hbm.at[pl.ds(base + lbase, _TPS // 2)])


def _sc_select(w1, u1, e1, n, cap):
    mesh = plsc.VectorSubcoreMesh(core_axis_name="c", subcore_axis_name="s")
    k = functools.partial(
        pl.kernel,
        mesh=mesh,
        compiler_params=pltpu.CompilerParams(needs_layout_passes=False),
        out_type=jax.ShapeDtypeStruct((n,), jnp.float32),
        scratch_types=[
            pltpu.VMEM((_TPS,), jnp.float32),
            pltpu.VMEM((_TPS,), jnp.int32),
            pltpu.VMEM((_TPS,), jnp.int32),
            pltpu.VMEM((_TPS,), jnp.int32),
            pltpu.VMEM((_TPS,), jnp.int32),
            pltpu.VMEM((16 * 1024,), jnp.int32),
            pltpu.VMEM((1024,), jnp.int32),
            pltpu.VMEM((16, 128), jnp.int32),
            pltpu.VMEM((16, 16), jnp.int32),
            pltpu.VMEM((16,), jnp.int32),
            pltpu.VMEM((16 * _E,), jnp.int32),
            pltpu.VMEM((_E,), jnp.int32),
            pltpu.VMEM((16 * _E,), jnp.int32),
            pltpu.VMEM((_E,), jnp.int32),
            pltpu.VMEM((_E,), jnp.int32),
            pltpu.VMEM((_E,), jnp.int32),
            pltpu.VMEM((_E,), jnp.int32),
            pltpu.VMEM((_E,), jnp.int32),
            pltpu.VMEM((_E,), jnp.int32),
            pltpu.VMEM((_TPS // 2,), jnp.float32),
            pltpu.SMEM((32,), jnp.int32),
            pltpu.VMEM_SHARED((2, 16, 1024), jnp.int32),
            pltpu.VMEM_SHARED((2, 16, 16), jnp.int32),
            pltpu.VMEM_SHARED((2 * 16 * _E,), jnp.int32),
        ],
    )(functools.partial(_sc_body, cap=cap, n=n))
    return k(w1, u1, e1)


def kernel(hidden_states, W_gate):
    b, s, d = hidden_states.shape
    n = b * s
    e = W_gate.shape[0]
    cap = int(n * _CAPF / e)
    chunk = 4096
    grid = n // chunk
    hs2 = hidden_states.reshape(n, d)

    w1, u1, e1, psum, cnt, aux = pl.pallas_call(
        functools.partial(_phase1_body, grid=grid),
        grid=(grid,),
        in_specs=[
            pl.BlockSpec((chunk, d), lambda i: (i, 0)),
            pl.BlockSpec((e, d), lambda i: (0, 0)),
        ],
        out_specs=[
            pl.BlockSpec((1, 1, chunk), lambda i: (i, 0, 0)),
            pl.BlockSpec((1, 1, chunk), lambda i: (i, 0, 0)),
            pl.BlockSpec((1, 1, chunk), lambda i: (i, 0, 0)),
            pl.BlockSpec((e, 128), lambda i: (0, 0)),
            pl.BlockSpec((e, 128), lambda i: (0, 0)),
            pl.BlockSpec((1, 1), lambda i: (0, 0)),
        ],
        out_shape=[
            jax.ShapeDtypeStruct((grid, 1, chunk), jnp.float32),
            jax.ShapeDtypeStruct((grid, 1, chunk), jnp.int32),
            jax.ShapeDtypeStruct((grid, 1, chunk), jnp.int32),
            jax.ShapeDtypeStruct((e, 128), jnp.float32),
            jax.ShapeDtypeStruct((e, 128), jnp.float32),
            jax.ShapeDtypeStruct((1, 1), jnp.float32),
        ],
    )(hs2, W_gate)

    wk = _sc_select(w1.reshape(n), u1.reshape(n), e1.reshape(n), n, cap)
    e1 = e1.reshape(n, 1)

    return (wk.reshape(n, 1), e1, cnt[:, 0], aux[0, 0])
